# baseline (speedup 1.0000x reference)
import functools

import jax
import jax.numpy as jnp
from jax import lax
from jax.experimental import pallas as pl
from jax.experimental.pallas import tpu as pltpu
from jax.experimental.pallas import tpu_sc as plsc

_NC = 2
_NS = 16
_W = _NC * _NS

_K = 128
_NB = 4
_BLK = 256


def _agg_call(n_pad, c_half, dh):
    rows_pt = n_pad // _NS
    mesh = plsc.VectorSubcoreMesh(core_axis_name="c", subcore_axis_name="s")

    @functools.partial(
        pl.kernel,
        out_type=jax.ShapeDtypeStruct((_NC, n_pad, dh), jnp.float32),
        mesh=mesh,
        compiler_params=pltpu.CompilerParams(use_tc_tiling_on_sc=False),
        scratch_types=[
            pltpu.VMEM((c_half, _K), jnp.int32),
            pltpu.VMEM((c_half, _K), jnp.int32),
            pltpu.VMEM((_K, dh), jnp.float32),
            pltpu.VMEM((_K, dh), jnp.float32),
            pltpu.VMEM((_K, dh), jnp.float32),
            pltpu.VMEM((_K, dh), jnp.float32),
            pltpu.VMEM_SHARED((n_pad, dh), jnp.float32),
            pltpu.SemaphoreType.DMA,
            pltpu.SemaphoreType.DMA,
            pltpu.SemaphoreType.DMA,
            pltpu.SemaphoreType.DMA,
            pltpu.SemaphoreType.DMA,
            pltpu.SemaphoreType.DMA,
            pltpu.SemaphoreType.DMA,
            pltpu.SemaphoreType.DMA,
        ],
    )
    def agg(g_hbm, srci_hbm, dsti_hbm, zeros_hbm, out_hbm,
            srci_v, dsti_v, b0, b1, b2, b3,
            acc, g0s, g1s, g2s, g3s, s0s, s1s, s2s, s3s):
        bufs = (b0, b1, b2, b3)
        gsems = (g0s, g1s, g2s, g3s)
        ssems = (s0s, s1s, s2s, s3s)
        cid = lax.axis_index("c")
        sid = lax.axis_index("s")
        pltpu.sync_copy(srci_hbm.at[sid], srci_v)
        pltpu.sync_copy(dsti_hbm.at[sid], dsti_v)
        pltpu.sync_copy(zeros_hbm, acc.at[pl.ds(sid * rows_pt, rows_pt)])
        plsc.subcore_barrier()

        ghalf = g_hbm.at[cid]

        @pl.loop(0, c_half, step=_NB)
        def _grp(jo):
            for k in range(_NB):
                pltpu.async_copy(ghalf.at[srci_v.at[jo + k]], bufs[k],
                                 gsems[k])
            for k in range(_NB):
                pltpu.make_async_copy(ghalf.at[srci_v.at[jo + k]], bufs[k],
                                      gsems[k]).wait()
                pltpu.async_copy(bufs[k], acc.at[dsti_v.at[jo + k]], ssems[k],
                                 add=True)
            for k in range(_NB):
                pltpu.make_async_copy(bufs[k], acc.at[dsti_v.at[jo + k]],
                                      ssems[k]).wait()

        plsc.subcore_barrier()
        pltpu.sync_copy(acc.at[pl.ds(sid * rows_pt, rows_pt)],
                        out_hbm.at[cid].at[pl.ds(sid * rows_pt, rows_pt)])

    return agg


def _deg_call(n_pad, c_chunks):
    rows_pt = n_pad // _NS
    mesh = plsc.VectorSubcoreMesh(core_axis_name="c", subcore_axis_name="s")

    @functools.partial(
        pl.kernel,
        out_type=jax.ShapeDtypeStruct((_NC, n_pad, 16), jnp.float32),
        mesh=mesh,
        compiler_params=pltpu.CompilerParams(use_tc_tiling_on_sc=False),
        scratch_types=[
            pltpu.VMEM((c_chunks, _K), jnp.int32),
            pltpu.VMEM((_K, 16), jnp.float32),
            pltpu.VMEM_SHARED((n_pad, 16), jnp.float32),
            pltpu.SemaphoreType.DMA,
        ],
    )
    def deg(dsti_hbm, ones_hbm, zeros_hbm, out_hbm, dsti_v, ones_v, acc, sem):
        cid = lax.axis_index("c")
        sid = lax.axis_index("s")
        wid = sid * _NC + cid
        pltpu.sync_copy(dsti_hbm.at[wid], dsti_v)
        pltpu.sync_copy(ones_hbm, ones_v)
        pltpu.sync_copy(zeros_hbm, acc.at[pl.ds(sid * rows_pt, rows_pt)])
        plsc.subcore_barrier()

        @pl.loop(0, c_chunks)
        def _chunk(j):
            pltpu.async_copy(ones_v, acc.at[dsti_v.at[j]], sem,
                             add=True).wait()

        plsc.subcore_barrier()
        pltpu.sync_copy(acc.at[pl.ds(sid * rows_pt, rows_pt)],
                        out_hbm.at[cid].at[pl.ds(sid * rows_pt, rows_pt)])

    return deg


def _tc_first(degparts, x_pad, w0_split):
    n_pad = x_pad.shape[0]
    dh = w0_split.shape[2]

    def body(dp_ref, x_ref, w_ref, g_ref, d_ref):
        dp = dp_ref[...]
        deg = dp[0, :, 0:1] + dp[1, :, 0:1] + 1.0
        dcol = lax.rsqrt(deg)
        x = x_ref[...]
        for c in range(_NC):
            g_ref[c] = dcol * jnp.dot(x, w_ref[c],
                                      preferred_element_type=jnp.float32)
        d_ref[...] = jnp.broadcast_to(dcol, (dcol.shape[0], dh))

    return pl.pallas_call(
        body,
        grid=(n_pad // _BLK,),
        in_specs=[
            pl.BlockSpec((2, _BLK, 16), lambda i: (0, i, 0)),
            pl.BlockSpec((_BLK, x_pad.shape[1]), lambda i: (i, 0)),
            pl.BlockSpec(w0_split.shape, lambda i: (0, 0, 0)),
        ],
        out_specs=[
            pl.BlockSpec((_NC, _BLK, dh), lambda i: (0, i, 0)),
            pl.BlockSpec((_BLK, dh), lambda i: (i, 0)),
        ],
        out_shape=[
            jax.ShapeDtypeStruct((_NC, n_pad, dh), jnp.float32),
            jax.ShapeDtypeStruct((n_pad, dh), jnp.float32),
        ],
    )(degparts, x_pad, w0_split)


def _tc_mid(parts, gsplit, d_half, b0, w1_split):
    _, n_pad, dh = gsplit.shape

    def body(p_ref, g_ref, d_ref, b_ref, w_ref, o_ref):
        d = d_ref[...]
        z = jnp.concatenate(
            [jnp.maximum(d * (p_ref[c] + g_ref[c]) + b_ref[c], 0.0)
             for c in range(_NC)], axis=1)
        for c in range(_NC):
            o_ref[c] = d * jnp.dot(z, w_ref[c],
                                   preferred_element_type=jnp.float32)

    sblk = pl.BlockSpec((_NC, _BLK, dh), lambda i: (0, i, 0))
    return pl.pallas_call(
        body,
        grid=(n_pad // _BLK,),
        in_specs=[sblk, sblk,
                  pl.BlockSpec((_BLK, dh), lambda i: (i, 0)),
                  pl.BlockSpec((_NC, 1, dh), lambda i: (0, 0, 0)),
                  pl.BlockSpec(w1_split.shape, lambda i: (0, 0, 0))],
        out_specs=sblk,
        out_shape=jax.ShapeDtypeStruct((_NC, n_pad, dh), jnp.float32),
    )(parts, gsplit, d_half, b0, w1_split)


def _tc_last(parts, gsplit, d_half, b1):
    _, n_pad, dh = gsplit.shape

    def body(p_ref, g_ref, d_ref, b_ref, o_ref):
        d = d_ref[...]
        o_ref[...] = jnp.concatenate(
            [jnp.maximum(d * (p_ref[c] + g_ref[c]) + b_ref[c], 0.0)
             for c in range(_NC)], axis=1)

    sblk = pl.BlockSpec((_NC, _BLK, dh), lambda i: (0, i, 0))
    return pl.pallas_call(
        body,
        grid=(n_pad // _BLK,),
        in_specs=[sblk, sblk,
                  pl.BlockSpec((_BLK, dh), lambda i: (i, 0)),
                  pl.BlockSpec((_NC, 1, dh), lambda i: (0, 0, 0))],
        out_specs=pl.BlockSpec((_BLK, _NC * dh), lambda i: (i, 0)),
        out_shape=jax.ShapeDtypeStruct((n_pad, _NC * dh), jnp.float32),
    )(parts, gsplit, d_half, b1)


def kernel(x, edge_index, W0, b0, W1, b1):
    n = x.shape[0]
    e = edge_index.shape[1]
    d_model = W0.shape[1]
    dh = d_model // _NC

    n_pad = ((n + 1 + _BLK - 1) // _BLK) * _BLK
    dummy = n

    per32 = -(-e // _W)
    c_deg = ((-(-per32 // _K) + 7) // 8) * 8
    e_pad = _W * c_deg * _K
    c_half = e_pad // (_NS * _K)

    src = edge_index[0].astype(jnp.int32)
    dst = edge_index[1].astype(jnp.int32)
    src_p = jnp.concatenate([src, jnp.zeros((e_pad - e,), jnp.int32)])
    dst_p = jnp.concatenate([dst, jnp.full((e_pad - e,), dummy, jnp.int32)])
    srci_h = src_p.reshape(_NS, c_half, _K)
    dsti_h = dst_p.reshape(_NS, c_half, _K)
    dsti_w = dst_p.reshape(_W, c_deg, _K)

    x_pad = jnp.pad(x, ((0, n_pad - n), (0, 0)))
    w0_split = W0.reshape(d_model, _NC, dh).transpose(1, 0, 2)
    w1_split = W1.reshape(d_model, _NC, dh).transpose(1, 0, 2)
    b0_split = b0.reshape(_NC, 1, dh)
    b1_split = b1.reshape(_NC, 1, dh)

    rows_pt = n_pad // _NS
    zeros_h = jnp.zeros((rows_pt, dh), jnp.float32)
    zeros_16 = jnp.zeros((rows_pt, 16), jnp.float32)
    ones_16 = jnp.ones((_K, 16), jnp.float32)

    degparts = _deg_call(n_pad, c_deg)(dsti_w, ones_16, zeros_16)
    g0, d_half = _tc_first(degparts, x_pad, w0_split)

    agg = _agg_call(n_pad, c_half, dh)
    parts0 = agg(g0, srci_h, dsti_h, zeros_h)
    g1 = _tc_mid(parts0, g0, d_half, b0_split, w1_split)
    parts1 = agg(g1, srci_h, dsti_h, zeros_h)
    out = _tc_last(parts1, g1, d_half, b1_split)
    return out[:n]


# speedup vs baseline: 1.0720x; 1.0720x over previous
import functools

import jax
import jax.numpy as jnp
from jax import lax
from jax.experimental import pallas as pl
from jax.experimental.pallas import tpu as pltpu
from jax.experimental.pallas import tpu_sc as plsc

_NC = 2
_NS = 16
_W = _NC * _NS

_K = 128
_BLK = 256


def _agg_call(n_pad, c_chunks, d_model):
    rows_pt = n_pad // _NS
    mesh = plsc.VectorSubcoreMesh(core_axis_name="c", subcore_axis_name="s")

    @functools.partial(
        pl.kernel,
        out_type=jax.ShapeDtypeStruct((_NC, n_pad, d_model), jnp.float32),
        mesh=mesh,
        compiler_params=pltpu.CompilerParams(use_tc_tiling_on_sc=False),
        scratch_types=[
            pltpu.VMEM((c_chunks, _K), jnp.int32),
            pltpu.VMEM((c_chunks, _K), jnp.int32),
            pltpu.VMEM((_K, d_model), jnp.float32),
            pltpu.VMEM_SHARED((n_pad, d_model), jnp.float32),
            pltpu.SemaphoreType.DMA,
            pltpu.SemaphoreType.DMA,
        ],
    )
    def agg(g_hbm, srci_hbm, dsti_hbm, zeros_hbm, out_hbm,
            srci_v, dsti_v, buf, acc, gsem, ssem):
        cid = lax.axis_index("c")
        sid = lax.axis_index("s")
        wid = sid * _NC + cid
        pltpu.sync_copy(srci_hbm.at[wid], srci_v)
        pltpu.sync_copy(dsti_hbm.at[wid], dsti_v)
        pltpu.sync_copy(zeros_hbm, acc.at[pl.ds(sid * rows_pt, rows_pt)])
        plsc.subcore_barrier()

        @pl.loop(0, c_chunks)
        def _chunk(j):
            pltpu.async_copy(g_hbm.at[srci_v.at[j]], buf, gsem).wait()
            pltpu.async_copy(buf, acc.at[dsti_v.at[j]], ssem, add=True).wait()

        plsc.subcore_barrier()
        pltpu.sync_copy(acc.at[pl.ds(sid * rows_pt, rows_pt)],
                        out_hbm.at[cid].at[pl.ds(sid * rows_pt, rows_pt)])

    return agg


def _deg_call(n_pad, c_chunks):
    rows_pt = n_pad // _NS
    mesh = plsc.VectorSubcoreMesh(core_axis_name="c", subcore_axis_name="s")

    @functools.partial(
        pl.kernel,
        out_type=jax.ShapeDtypeStruct((_NC, n_pad, 16), jnp.float32),
        mesh=mesh,
        compiler_params=pltpu.CompilerParams(use_tc_tiling_on_sc=False),
        scratch_types=[
            pltpu.VMEM((c_chunks, _K), jnp.int32),
            pltpu.VMEM((_K, 16), jnp.float32),
            pltpu.VMEM_SHARED((n_pad, 16), jnp.float32),
            pltpu.SemaphoreType.DMA,
        ],
    )
    def deg(dsti_hbm, ones_hbm, zeros_hbm, out_hbm, dsti_v, ones_v, acc, sem):
        cid = lax.axis_index("c")
        sid = lax.axis_index("s")
        wid = sid * _NC + cid
        pltpu.sync_copy(dsti_hbm.at[wid], dsti_v)
        pltpu.sync_copy(ones_hbm, ones_v)
        pltpu.sync_copy(zeros_hbm, acc.at[pl.ds(sid * rows_pt, rows_pt)])
        plsc.subcore_barrier()

        @pl.loop(0, c_chunks)
        def _chunk(j):
            pltpu.async_copy(ones_v, acc.at[dsti_v.at[j]], sem,
                             add=True).wait()

        plsc.subcore_barrier()
        pltpu.sync_copy(acc.at[pl.ds(sid * rows_pt, rows_pt)],
                        out_hbm.at[cid].at[pl.ds(sid * rows_pt, rows_pt)])

    return deg


def _tc_first(degparts, x_pad, w0):
    n_pad = x_pad.shape[0]
    d_model = w0.shape[1]

    def body(dp_ref, x_ref, w_ref, g_ref, d_ref):
        dp = dp_ref[...]
        deg = dp[0, :, 0:1] + dp[1, :, 0:1] + 1.0
        dcol = lax.rsqrt(deg)
        h = jnp.dot(x_ref[...], w_ref[...], preferred_element_type=jnp.float32)
        g_ref[...] = dcol * h
        d_ref[...] = jnp.broadcast_to(dcol, (dcol.shape[0], d_model))

    return pl.pallas_call(
        body,
        grid=(n_pad // _BLK,),
        in_specs=[
            pl.BlockSpec((2, _BLK, 16), lambda i: (0, i, 0)),
            pl.BlockSpec((_BLK, x_pad.shape[1]), lambda i: (i, 0)),
            pl.BlockSpec(w0.shape, lambda i: (0, 0)),
        ],
        out_specs=[
            pl.BlockSpec((_BLK, d_model), lambda i: (i, 0)),
            pl.BlockSpec((_BLK, d_model), lambda i: (i, 0)),
        ],
        out_shape=[
            jax.ShapeDtypeStruct((n_pad, d_model), jnp.float32),
            jax.ShapeDtypeStruct((n_pad, d_model), jnp.float32),
        ],
    )(degparts, x_pad, w0)


def _tc_mid(parts, g0, d_bcast, b0, w1):
    _, n_pad, d_model = parts.shape

    def body(p_ref, g_ref, d_ref, b_ref, w_ref, o_ref):
        acc = p_ref[0] + p_ref[1] + g_ref[...]
        d = d_ref[...]
        z = jnp.maximum(d * acc + b_ref[...], 0.0)
        h = jnp.dot(z, w_ref[...], preferred_element_type=jnp.float32)
        o_ref[...] = d * h

    blk = pl.BlockSpec((_BLK, d_model), lambda i: (i, 0))
    return pl.pallas_call(
        body,
        grid=(n_pad // _BLK,),
        in_specs=[pl.BlockSpec((_NC, _BLK, d_model), lambda i: (0, i, 0)),
                  blk, blk,
                  pl.BlockSpec((1, d_model), lambda i: (0, 0)),
                  pl.BlockSpec(w1.shape, lambda i: (0, 0))],
        out_specs=blk,
        out_shape=jax.ShapeDtypeStruct((n_pad, d_model), jnp.float32),
    )(parts, g0, d_bcast, b0, w1)


def _tc_last(parts, g1, d_bcast, b1):
    _, n_pad, d_model = parts.shape

    def body(p_ref, g_ref, d_ref, b_ref, o_ref):
        acc = p_ref[0] + p_ref[1] + g_ref[...]
        o_ref[...] = jnp.maximum(d_ref[...] * acc + b_ref[...], 0.0)

    blk = pl.BlockSpec((_BLK, d_model), lambda i: (i, 0))
    return pl.pallas_call(
        body,
        grid=(n_pad // _BLK,),
        in_specs=[pl.BlockSpec((_NC, _BLK, d_model), lambda i: (0, i, 0)),
                  blk, blk,
                  pl.BlockSpec((1, d_model), lambda i: (0, 0))],
        out_specs=blk,
        out_shape=jax.ShapeDtypeStruct((n_pad, d_model), jnp.float32),
    )(parts, g1, d_bcast, b1)


def kernel(x, edge_index, W0, b0, W1, b1):
    n = x.shape[0]
    e = edge_index.shape[1]
    d_model = W0.shape[1]

    n_pad = ((n + 1 + _BLK - 1) // _BLK) * _BLK
    dummy = n

    per_tile = -(-e // _W)
    c_chunks = -(-per_tile // _K)
    e_pad = _W * c_chunks * _K

    src = edge_index[0].astype(jnp.int32)
    dst = edge_index[1].astype(jnp.int32)
    src_p = jnp.concatenate([src, jnp.zeros((e_pad - e,), jnp.int32)])
    dst_p = jnp.concatenate([dst, jnp.full((e_pad - e,), dummy, jnp.int32)])
    srci = src_p.reshape(_W, c_chunks, _K)
    dsti = dst_p.reshape(_W, c_chunks, _K)

    x_pad = jnp.pad(x, ((0, n_pad - n), (0, 0)))
    rows_pt = n_pad // _NS
    zeros_wide = jnp.zeros((rows_pt, d_model), jnp.float32)
    zeros_16 = jnp.zeros((rows_pt, 16), jnp.float32)
    ones_16 = jnp.ones((_K, 16), jnp.float32)

    degparts = _deg_call(n_pad, c_chunks)(dsti, ones_16, zeros_16)
    g0, d_bcast = _tc_first(degparts, x_pad, W0)

    agg = _agg_call(n_pad, c_chunks, d_model)
    parts0 = agg(g0, srci, dsti, zeros_wide)
    g1 = _tc_mid(parts0, g0, d_bcast, b0.reshape(1, -1), W1)
    parts1 = agg(g1, srci, dsti, zeros_wide)
    out = _tc_last(parts1, g1, d_bcast, b1.reshape(1, -1))
    return out[:n]


# speedup vs baseline: 1.0952x; 1.0216x over previous
import functools

import jax
import jax.numpy as jnp
from jax import lax
from jax.experimental import pallas as pl
from jax.experimental.pallas import tpu as pltpu
from jax.experimental.pallas import tpu_sc as plsc

_NC = 2
_NS = 16
_W = _NC * _NS

_K = 128
_BLK = 256


def _agg_call(n_pad, c_chunks, d_model):
    rows_pt = n_pad // _NS
    mesh = plsc.VectorSubcoreMesh(core_axis_name="c", subcore_axis_name="s")

    @functools.partial(
        pl.kernel,
        out_type=jax.ShapeDtypeStruct((_NC, n_pad, d_model), jnp.float32),
        mesh=mesh,
        compiler_params=pltpu.CompilerParams(use_tc_tiling_on_sc=False),
        scratch_types=[
            pltpu.VMEM((c_chunks, _K), jnp.int32),
            pltpu.VMEM((c_chunks, _K), jnp.int32),
            pltpu.VMEM((_K, d_model), jnp.float32),
            pltpu.VMEM_SHARED((n_pad, d_model), jnp.float32),
            pltpu.SemaphoreType.DMA,
            pltpu.SemaphoreType.DMA,
        ],
    )
    def agg(g_hbm, srci_hbm, dsti_hbm, zeros_hbm, out_hbm,
            srci_v, dsti_v, buf, acc, gsem, ssem):
        cid = lax.axis_index("c")
        sid = lax.axis_index("s")
        wid = sid * _NC + cid
        pltpu.sync_copy(srci_hbm.at[wid], srci_v)
        pltpu.sync_copy(dsti_hbm.at[wid], dsti_v)
        pltpu.sync_copy(zeros_hbm, acc.at[pl.ds(sid * rows_pt, rows_pt)])
        plsc.subcore_barrier()

        ghalf = g_hbm.at[cid]

        @pl.loop(0, c_chunks)
        def _chunk(j):
            pltpu.async_copy(ghalf.at[srci_v.at[j]], buf, gsem).wait()
            pltpu.async_copy(buf, acc.at[dsti_v.at[j]], ssem, add=True).wait()

        plsc.subcore_barrier()
        pltpu.sync_copy(acc.at[pl.ds(sid * rows_pt, rows_pt)],
                        out_hbm.at[cid].at[pl.ds(sid * rows_pt, rows_pt)])

    return agg


def _deg_call(n_pad, c_chunks):
    rows_pt = n_pad // _NS
    mesh = plsc.VectorSubcoreMesh(core_axis_name="c", subcore_axis_name="s")

    @functools.partial(
        pl.kernel,
        out_type=jax.ShapeDtypeStruct((_NC, n_pad, 16), jnp.float32),
        mesh=mesh,
        compiler_params=pltpu.CompilerParams(use_tc_tiling_on_sc=False),
        scratch_types=[
            pltpu.VMEM((c_chunks, _K), jnp.int32),
            pltpu.VMEM((_K, 16), jnp.float32),
            pltpu.VMEM_SHARED((n_pad, 16), jnp.float32),
            pltpu.SemaphoreType.DMA,
        ],
    )
    def deg(dsti_hbm, ones_hbm, zeros_hbm, out_hbm, dsti_v, ones_v, acc, sem):
        cid = lax.axis_index("c")
        sid = lax.axis_index("s")
        wid = sid * _NC + cid
        pltpu.sync_copy(dsti_hbm.at[wid], dsti_v)
        pltpu.sync_copy(ones_hbm, ones_v)
        pltpu.sync_copy(zeros_hbm, acc.at[pl.ds(sid * rows_pt, rows_pt)])
        plsc.subcore_barrier()

        @pl.loop(0, c_chunks)
        def _chunk(j):
            pltpu.async_copy(ones_v, acc.at[dsti_v.at[j]], sem,
                             add=True).wait()

        plsc.subcore_barrier()
        pltpu.sync_copy(acc.at[pl.ds(sid * rows_pt, rows_pt)],
                        out_hbm.at[cid].at[pl.ds(sid * rows_pt, rows_pt)])

    return deg


def _tc_first(degparts, x_pad, w0):
    n_pad = x_pad.shape[0]
    d_model = w0.shape[1]

    def body(dp_ref, x_ref, w_ref, g_ref, d_ref):
        dp = dp_ref[...]
        deg = dp[0, :, 0:1] + dp[1, :, 0:1] + 1.0
        dcol = lax.rsqrt(deg)
        h = jnp.dot(x_ref[...], w_ref[...], preferred_element_type=jnp.float32)
        gh = dcol * h
        g_ref[0] = gh
        g_ref[1] = gh
        d_ref[...] = jnp.broadcast_to(dcol, (dcol.shape[0], d_model))

    return pl.pallas_call(
        body,
        grid=(n_pad // _BLK,),
        in_specs=[
            pl.BlockSpec((2, _BLK, 16), lambda i: (0, i, 0)),
            pl.BlockSpec((_BLK, x_pad.shape[1]), lambda i: (i, 0)),
            pl.BlockSpec(w0.shape, lambda i: (0, 0)),
        ],
        out_specs=[
            pl.BlockSpec((_NC, _BLK, d_model), lambda i: (0, i, 0)),
            pl.BlockSpec((_BLK, d_model), lambda i: (i, 0)),
        ],
        out_shape=[
            jax.ShapeDtypeStruct((_NC, n_pad, d_model), jnp.float32),
            jax.ShapeDtypeStruct((n_pad, d_model), jnp.float32),
        ],
    )(degparts, x_pad, w0)


def _tc_mid(parts, g0, d_bcast, b0, w1):
    _, n_pad, d_model = parts.shape

    def body(p_ref, g_ref, d_ref, b_ref, w_ref, o_ref):
        acc = p_ref[0] + p_ref[1] + g_ref[0]
        d = d_ref[...]
        z = jnp.maximum(d * acc + b_ref[...], 0.0)
        h = jnp.dot(z, w_ref[...], preferred_element_type=jnp.float32)
        g1 = d * h
        o_ref[0] = g1
        o_ref[1] = g1

    blk = pl.BlockSpec((_BLK, d_model), lambda i: (i, 0))
    dblk = pl.BlockSpec((_NC, _BLK, d_model), lambda i: (0, i, 0))
    return pl.pallas_call(
        body,
        grid=(n_pad // _BLK,),
        in_specs=[dblk, dblk, blk,
                  pl.BlockSpec((1, d_model), lambda i: (0, 0)),
                  pl.BlockSpec(w1.shape, lambda i: (0, 0))],
        out_specs=dblk,
        out_shape=jax.ShapeDtypeStruct((_NC, n_pad, d_model), jnp.float32),
    )(parts, g0, d_bcast, b0, w1)


def _tc_last(parts, g1, d_bcast, b1):
    _, n_pad, d_model = parts.shape

    def body(p_ref, g_ref, d_ref, b_ref, o_ref):
        acc = p_ref[0] + p_ref[1] + g_ref[0]
        o_ref[...] = jnp.maximum(d_ref[...] * acc + b_ref[...], 0.0)

    blk = pl.BlockSpec((_BLK, d_model), lambda i: (i, 0))
    dblk = pl.BlockSpec((_NC, _BLK, d_model), lambda i: (0, i, 0))
    return pl.pallas_call(
        body,
        grid=(n_pad // _BLK,),
        in_specs=[dblk, dblk, blk,
                  pl.BlockSpec((1, d_model), lambda i: (0, 0))],
        out_specs=blk,
        out_shape=jax.ShapeDtypeStruct((n_pad, d_model), jnp.float32),
    )(parts, g1, d_bcast, b1)


def kernel(x, edge_index, W0, b0, W1, b1):
    n = x.shape[0]
    e = edge_index.shape[1]
    d_model = W0.shape[1]

    n_pad = ((n + 1 + _BLK - 1) // _BLK) * _BLK
    dummy = n

    per_tile = -(-e // _W)
    c_chunks = -(-per_tile // _K)
    e_pad = _W * c_chunks * _K

    src = edge_index[0].astype(jnp.int32)
    dst = edge_index[1].astype(jnp.int32)
    src_p = jnp.concatenate([src, jnp.zeros((e_pad - e,), jnp.int32)])
    dst_p = jnp.concatenate([dst, jnp.full((e_pad - e,), dummy, jnp.int32)])
    srci = src_p.reshape(_W, c_chunks, _K)
    dsti = dst_p.reshape(_W, c_chunks, _K)

    x_pad = jnp.pad(x, ((0, n_pad - n), (0, 0)))
    rows_pt = n_pad // _NS
    zeros_wide = jnp.zeros((rows_pt, d_model), jnp.float32)
    zeros_16 = jnp.zeros((rows_pt, 16), jnp.float32)
    ones_16 = jnp.ones((_K, 16), jnp.float32)

    degparts = _deg_call(n_pad, c_chunks)(dsti, ones_16, zeros_16)
    g0, d_bcast = _tc_first(degparts, x_pad, W0)

    agg = _agg_call(n_pad, c_chunks, d_model)
    parts0 = agg(g0, srci, dsti, zeros_wide)
    g1 = _tc_mid(parts0, g0, d_bcast, b0.reshape(1, -1), W1)
    parts1 = agg(g1, srci, dsti, zeros_wide)
    out = _tc_last(parts1, g1, d_bcast, b1.reshape(1, -1))
    return out[:n]
